# baseline (speedup 1.0000x reference)
import functools
import math

import numpy as np
import jax
import jax.numpy as jnp
from jax.experimental import pallas as pl
from jax.experimental.pallas import tpu as pltpu


def _band(num_w: int, num_p: int, wp: int) -> np.ndarray:
    w = np.arange(num_w)[:, None, None]
    kj = np.arange(5)[None, :, None]
    p = np.arange(num_p)[None, None, :]
    return (w == 2 * p + wp + kj).astype(np.float32)


def _net_kernel(x_ref, w1_ref, b1_ref, w2_ref, b2_ref, wf1a_ref, wf1b_ref,
                bf1_ref, wf2_ref, bf2_ref, wf3_ref, bf3_ref, o_ref):
    def mm(a, w):
        return jax.lax.dot_general(a, w, (((1,), (0,)), ((), ())),
                                   preferred_element_type=jnp.float32)

    def conv_row(oh, w_ref, rws):
        par, base = oh % 2, oh // 2
        return (mm(rws[base], w_ref[par * 3 + 0])
                + mm(rws[base + 1], w_ref[par * 3 + 1])
                + mm(rws[base + 2], w_ref[par * 3 + 2]))

    def pool(ya, yb, bias):
        m = jnp.maximum(jnp.maximum(ya[:, :128], ya[:, 128:]),
                        jnp.maximum(yb[:, :128], yb[:, 128:]))
        return jnp.maximum(m + bias, 0.0)

    def conv1_row(oh):
        q, a = oh % 4, oh // 4
        acc = mm(x_ref[:, a, :], w1_ref[q * 6 + 0])
        acc += mm(x_ref[:, a + 1, :], w1_ref[q * 6 + 3])
        for c in (1, 2):
            acc += mm(x_ref[:, c * 8 + a, :], w1_ref[q * 6 + c])
            acc += mm(x_ref[:, c * 8 + a + 1, :], w1_ref[q * 6 + 3 + c])
        return acc

    b1 = b1_ref[...]
    p1 = [pool(conv1_row(2 * ph), conv1_row(2 * ph + 1), b1)
          for ph in range(14)]

    pairs = [jnp.concatenate([p1[2 * j], p1[2 * j + 1]], axis=1)
             for j in range(7)]

    b2 = b2_ref[...]
    p2 = [pool(conv_row(2 * ph2, w2_ref, pairs),
               conv_row(2 * ph2 + 1, w2_ref, pairs), b2)
          for ph2 in range(5)]

    q01 = jnp.concatenate([p2[0], p2[1]], axis=1)
    q23 = jnp.concatenate([p2[2], p2[3]], axis=1)
    h = mm(q01, wf1a_ref[0]) + mm(q23, wf1a_ref[1]) + mm(p2[4], wf1b_ref[...])
    h = jnp.maximum(h + bf1_ref[...], 0.0)
    h = jnp.maximum(mm(h, wf2_ref[...]) + bf2_ref[...], 0.0)
    o_ref[...] = mm(h, wf3_ref[...]) + bf3_ref[...]


_PAIR_IDX = np.array([[0, 1], [2, 3], [4, 5],
                      [5, 0], [1, 2], [3, 4]])


def _banded(ind, w, rows, cols):
    m = jnp.einsum('awkp,ikco->aiwcpo', jnp.asarray(ind), w)
    m = m.reshape(2, 5, m.shape[2] * m.shape[3], -1)
    m = jnp.pad(m, ((0, 0), (0, 1), (0, rows - m.shape[2]),
                    (0, cols // 2 - m.shape[3])))
    m = m.transpose(1, 2, 0, 3).reshape(6, rows, cols)
    return m[_PAIR_IDX].reshape(6, 2 * rows, cols)


def _pack_weights(conv1_w, conv1_b, conv2_w, conv2_b, fc1_w, fc1_b):
    f32 = jnp.float32

    w1 = conv1_w.reshape(5, 5, 3, 128)[:, :, :, :8]
    ind1 = np.stack([_band(32, 14, 0), _band(32, 14, 1)])
    m = jnp.einsum('awkp,ikco->aiwcpo', jnp.asarray(ind1), w1)
    q = np.arange(4)[:, None, None, None]
    v = np.arange(2)[None, :, None, None]
    hr = np.arange(4)[None, None, :, None]
    ki = np.arange(5)[None, None, None, :]
    sel = (ki == hr - q + 4 * v).astype(np.float32)
    w1s = jnp.einsum('qvhi,aiwcpo->qvchwapo', jnp.asarray(sel), m)
    w1s = w1s.reshape(4, 2, 3, 128, 2, 112)
    w1s = jnp.pad(w1s, ((0, 0),) * 5 + ((0, 16),)).reshape(24, 128, 256)

    w2 = conv2_w.reshape(5, 5, 6, 128)[:, :, :, :16]
    w2p = jnp.pad(w2, ((0, 0), (0, 0), (0, 2), (0, 0)))
    ind2 = np.stack([_band(14, 5, 0), _band(14, 5, 1)])
    w2s = _banded(ind2, w2p, 128, 256)

    b1t = jnp.concatenate([jnp.tile(conv1_b[:, :8], (1, 14)),
                           jnp.zeros((1, 16), f32)], axis=1)
    b2t = jnp.concatenate([jnp.tile(conv2_b[:, :16], (1, 5)),
                           jnp.zeros((1, 48), f32)], axis=1)

    fr = fc1_w.reshape(5, 5, 128, 128)[:, :, :16, :]
    F = [jnp.pad(fr[p].reshape(80, 128), ((0, 48), (0, 0))) for p in range(5)]
    wf1a = jnp.stack([jnp.concatenate([F[0], F[1]], 0),
                      jnp.concatenate([F[2], F[3]], 0)])
    wf1b = F[4]
    return w1s, w2s, b1t, b2t, wf1a, wf1b


def kernel(conv1_w, conv1_b, conv2_w, conv2_b, fc1_w, fc1_b,
           fc2_w, fc2_b, fc3_w, fc3_b, x):
    N = x.shape[0]
    w1s, w2s, b1t, b2t, wf1a, wf1b = _pack_weights(
        conv1_w, conv1_b, conv2_w, conv2_b, fc1_w, fc1_b)

    x2 = x.reshape(N, 24, 128)

    BB = 256
    while N % BB:
        BB //= 2
    grid = (N // BB,)

    full = lambda shape: pl.BlockSpec(shape, lambda i: tuple(0 for _ in shape))
    out = pl.pallas_call(
        _net_kernel,
        out_shape=jax.ShapeDtypeStruct((N, 128), jnp.float32),
        grid=grid,
        in_specs=[pl.BlockSpec((BB, 24, 128), lambda i: (i, 0, 0)),
                  full((24, 128, 256)), full((1, 128)),
                  full((6, 256, 256)), full((1, 128)),
                  full((2, 256, 128)), full((128, 128)), full((1, 128)),
                  full((128, 128)), full((1, 128)),
                  full((128, 128)), full((1, 128))],
        out_specs=pl.BlockSpec((BB, 128), lambda i: (i, 0)),
        compiler_params=pltpu.CompilerParams(
            dimension_semantics=("parallel",)),
    )(x2, w1s, b1t, w2s, b2t, wf1a, wf1b, fc1_b, fc2_w, fc2_b,
      fc3_w, fc3_b)
    return out[:, :10]


# speedup vs baseline: 1.2165x; 1.2165x over previous
import functools
import math

import numpy as np
import jax
import jax.numpy as jnp
from jax.experimental import pallas as pl
from jax.experimental.pallas import tpu as pltpu


def _band(num_w: int, num_p: int, wp: int) -> np.ndarray:
    w = np.arange(num_w)[:, None, None]
    kj = np.arange(5)[None, :, None]
    p = np.arange(num_p)[None, None, :]
    return (w == 2 * p + wp + kj).astype(np.float32)


def _net_kernel(x_ref, w1_ref, b1_ref, w2_ref, b2_ref, wf1a_ref, wf1b_ref,
                bf1_ref, wf2_ref, bf2_ref, wf3_ref, bf3_ref, o_ref):
    def mm(a, w):
        return jax.lax.dot_general(a, w, (((1,), (0,)), ((), ())),
                                   preferred_element_type=jnp.float32)

    def conv_row(oh, w_ref, rws):
        par, base = oh % 2, oh // 2
        return (mm(rws[base], w_ref[par * 3 + 0])
                + mm(rws[base + 1], w_ref[par * 3 + 1])
                + mm(rws[base + 2], w_ref[par * 3 + 2]))

    def pool(ya, yb, bias):
        m = jnp.maximum(jnp.maximum(ya[:, :128], ya[:, 128:]),
                        jnp.maximum(yb[:, :128], yb[:, 128:]))
        return jnp.maximum(m + bias, 0.0)

    def conv1_row(oh):
        q, a = oh % 4, oh // 4
        acc = None
        for c in range(3):
            lhs = x_ref[:, c * 8 + a: c * 8 + a + 2, :]
            lhs = lhs.reshape(lhs.shape[0], 256)
            t = mm(lhs, w1_ref[q * 3 + c])
            acc = t if acc is None else acc + t
        return acc

    b1 = b1_ref[...]
    p1 = [pool(conv1_row(2 * ph), conv1_row(2 * ph + 1), b1)
          for ph in range(14)]

    pairs = [jnp.concatenate([p1[2 * j], p1[2 * j + 1]], axis=1)
             for j in range(7)]

    b2 = b2_ref[...]
    p2 = [pool(conv_row(2 * ph2, w2_ref, pairs),
               conv_row(2 * ph2 + 1, w2_ref, pairs), b2)
          for ph2 in range(5)]

    q01 = jnp.concatenate([p2[0], p2[1]], axis=1)
    q23 = jnp.concatenate([p2[2], p2[3]], axis=1)
    h = mm(q01, wf1a_ref[0]) + mm(q23, wf1a_ref[1]) + mm(p2[4], wf1b_ref[...])
    h = jnp.maximum(h + bf1_ref[...], 0.0)
    h = jnp.maximum(mm(h, wf2_ref[...]) + bf2_ref[...], 0.0)
    o_ref[...] = mm(h, wf3_ref[...]) + bf3_ref[...]


_PAIR_IDX = np.array([[0, 1], [2, 3], [4, 5],
                      [5, 0], [1, 2], [3, 4]])


def _banded(ind, w, rows, cols):
    m = jnp.einsum('awkp,ikco->aiwcpo', jnp.asarray(ind), w)
    m = m.reshape(2, 5, m.shape[2] * m.shape[3], -1)
    m = jnp.pad(m, ((0, 0), (0, 1), (0, rows - m.shape[2]),
                    (0, cols // 2 - m.shape[3])))
    m = m.transpose(1, 2, 0, 3).reshape(6, rows, cols)
    return m[_PAIR_IDX].reshape(6, 2 * rows, cols)


def _pack_weights(conv1_w, conv1_b, conv2_w, conv2_b, fc1_w, fc1_b):
    f32 = jnp.float32

    w1 = conv1_w.reshape(5, 5, 3, 128)[:, :, :, :8]
    ind1 = np.stack([_band(32, 14, 0), _band(32, 14, 1)])
    m = jnp.einsum('awkp,ikco->aiwcpo', jnp.asarray(ind1), w1)
    q = np.arange(4)[:, None, None, None]
    v = np.arange(2)[None, :, None, None]
    hr = np.arange(4)[None, None, :, None]
    ki = np.arange(5)[None, None, None, :]
    sel = (ki == hr - q + 4 * v).astype(np.float32)
    w1s = jnp.einsum('qvhi,aiwcpo->qcvhwapo', jnp.asarray(sel), m)
    w1s = w1s.reshape(4, 3, 256, 2, 112)
    w1s = jnp.pad(w1s, ((0, 0),) * 4 + ((0, 16),)).reshape(12, 256, 256)

    w2 = conv2_w.reshape(5, 5, 6, 128)[:, :, :, :16]
    w2p = jnp.pad(w2, ((0, 0), (0, 0), (0, 2), (0, 0)))
    ind2 = np.stack([_band(14, 5, 0), _band(14, 5, 1)])
    w2s = _banded(ind2, w2p, 128, 256)

    b1t = jnp.concatenate([jnp.tile(conv1_b[:, :8], (1, 14)),
                           jnp.zeros((1, 16), f32)], axis=1)
    b2t = jnp.concatenate([jnp.tile(conv2_b[:, :16], (1, 5)),
                           jnp.zeros((1, 48), f32)], axis=1)

    fr = fc1_w.reshape(5, 5, 128, 128)[:, :, :16, :]
    F = [jnp.pad(fr[p].reshape(80, 128), ((0, 48), (0, 0))) for p in range(5)]
    wf1a = jnp.stack([jnp.concatenate([F[0], F[1]], 0),
                      jnp.concatenate([F[2], F[3]], 0)])
    wf1b = F[4]
    return w1s, w2s, b1t, b2t, wf1a, wf1b


def kernel(conv1_w, conv1_b, conv2_w, conv2_b, fc1_w, fc1_b,
           fc2_w, fc2_b, fc3_w, fc3_b, x):
    N = x.shape[0]
    w1s, w2s, b1t, b2t, wf1a, wf1b = _pack_weights(
        conv1_w, conv1_b, conv2_w, conv2_b, fc1_w, fc1_b)

    x2 = x.reshape(N, 24, 128)

    BB = 256
    while N % BB:
        BB //= 2
    grid = (N // BB,)

    full = lambda shape: pl.BlockSpec(shape, lambda i: tuple(0 for _ in shape))
    out = pl.pallas_call(
        _net_kernel,
        out_shape=jax.ShapeDtypeStruct((N, 128), jnp.float32),
        grid=grid,
        in_specs=[pl.BlockSpec((BB, 24, 128), lambda i: (i, 0, 0)),
                  full((12, 256, 256)), full((1, 128)),
                  full((6, 256, 256)), full((1, 128)),
                  full((2, 256, 128)), full((128, 128)), full((1, 128)),
                  full((128, 128)), full((1, 128)),
                  full((128, 128)), full((1, 128))],
        out_specs=pl.BlockSpec((BB, 128), lambda i: (i, 0)),
        compiler_params=pltpu.CompilerParams(
            dimension_semantics=("parallel",)),
    )(x2, w1s, b1t, w2s, b2t, wf1a, wf1b, fc1_b, fc2_w, fc2_b,
      fc3_w, fc3_b)
    return out[:, :10]


# speedup vs baseline: 1.8330x; 1.5067x over previous
import functools
import math

import numpy as np
import jax
import jax.numpy as jnp
from jax.experimental import pallas as pl
from jax.experimental.pallas import tpu as pltpu


def _band(num_w: int, num_p: int, wp: int) -> np.ndarray:
    w = np.arange(num_w)[:, None, None]
    kj = np.arange(5)[None, :, None]
    p = np.arange(num_p)[None, None, :]
    return (w == 2 * p + wp + kj).astype(np.float32)


def _net_kernel(x_ref, w1_ref, b1_ref, w2_ref, b2_ref, wf1a_ref, wf1b_ref,
                bf1_ref, wf2_ref, bf2_ref, wf3_ref, bf3_ref, o_ref):
    def mm(a, w):
        return jax.lax.dot_general(a, w, (((1,), (0,)), ((), ())),
                                   preferred_element_type=jnp.float32)

    def conv2_row(oh2, rws):
        par, base = oh2 % 2, oh2 // 2
        start = 128 * (1 - par)
        acc = None
        for t in range(3):
            rhs = w2_ref[start + 256 * t: start + 256 * t + 256, :]
            m = mm(rws[base + t], rhs)
            acc = m if acc is None else acc + m
        return acc

    def pool(ya, yb, bias):
        m = jnp.maximum(jnp.maximum(ya[:, :128], ya[:, 128:]),
                        jnp.maximum(yb[:, :128], yb[:, 128:]))
        return jnp.maximum(m + bias, 0.0)

    def conv1_row(oh):
        q, a = oh % 4, oh // 4
        acc = None
        for c in range(3):
            lhs = x_ref[:, c * 8 + a: c * 8 + a + 2, :]
            lhs = lhs.reshape(lhs.shape[0], 256)
            t = mm(lhs, w1_ref[c, 96 - 32 * q: 352 - 32 * q, :])
            acc = t if acc is None else acc + t
        return acc

    b1 = b1_ref[...]
    p1 = [pool(conv1_row(2 * ph), conv1_row(2 * ph + 1), b1)
          for ph in range(14)]

    pairs = [jnp.concatenate([p1[2 * j], p1[2 * j + 1]], axis=1)
             for j in range(7)]

    b2 = b2_ref[...]
    p2 = [pool(conv2_row(2 * ph2, pairs),
               conv2_row(2 * ph2 + 1, pairs), b2)
          for ph2 in range(5)]

    q01 = jnp.concatenate([p2[0], p2[1]], axis=1)
    q23 = jnp.concatenate([p2[2], p2[3]], axis=1)
    h = mm(q01, wf1a_ref[0]) + mm(q23, wf1a_ref[1]) + mm(p2[4], wf1b_ref[...])
    h = jnp.maximum(h + bf1_ref[...], 0.0)
    h = jnp.maximum(mm(h, wf2_ref[...]) + bf2_ref[...], 0.0)
    o_ref[...] = (mm(h, wf3_ref[...]) + bf3_ref[...])[:, :10]


def _pack_weights(conv1_w, conv1_b, conv2_w, conv2_b, fc1_w, fc1_b):
    f32 = jnp.float32

    w1 = conv1_w.reshape(5, 5, 3, 128)[:, :, :, :8]
    ind1 = np.stack([_band(32, 14, 0), _band(32, 14, 1)])
    w1s = jnp.einsum('awkp,ikco->ciwapo', jnp.asarray(ind1), w1)
    w1s = w1s.reshape(3, 160, 2, 112)
    w1s = jnp.pad(w1s, ((0, 0), (96, 96), (0, 0), (0, 16)))
    w1s = w1s.reshape(3, 352, 256)

    w2 = conv2_w.reshape(5, 5, 6, 128)[:, :, :, :16]
    w2p = jnp.pad(w2, ((0, 0), (0, 0), (0, 2), (0, 0)))
    ind2 = np.stack([_band(14, 5, 0), _band(14, 5, 1)])
    w2s = jnp.einsum('awkp,ikco->iwcapo', jnp.asarray(ind2), w2p)
    w2s = w2s.reshape(5, 14, 8, 2, 80)
    w2s = jnp.pad(w2s, ((0, 0), (0, 2), (0, 0), (0, 0), (0, 48)))
    w2s = w2s.reshape(5, 128, 256)
    w2s = jnp.pad(w2s, ((1, 1), (0, 0), (0, 0))).reshape(896, 256)

    b1t = jnp.concatenate([jnp.tile(conv1_b[:, :8], (1, 14)),
                           jnp.zeros((1, 16), f32)], axis=1)
    b2t = jnp.concatenate([jnp.tile(conv2_b[:, :16], (1, 5)),
                           jnp.zeros((1, 48), f32)], axis=1)

    fr = fc1_w.reshape(5, 5, 128, 128)[:, :, :16, :]
    F = [jnp.pad(fr[p].reshape(80, 128), ((0, 48), (0, 0))) for p in range(5)]
    wf1a = jnp.stack([jnp.concatenate([F[0], F[1]], 0),
                      jnp.concatenate([F[2], F[3]], 0)])
    wf1b = F[4]
    return w1s, w2s, b1t, b2t, wf1a, wf1b


def kernel(conv1_w, conv1_b, conv2_w, conv2_b, fc1_w, fc1_b,
           fc2_w, fc2_b, fc3_w, fc3_b, x):
    N = x.shape[0]
    w1s, w2s, b1t, b2t, wf1a, wf1b = _pack_weights(
        conv1_w, conv1_b, conv2_w, conv2_b, fc1_w, fc1_b)

    x2 = x.reshape(N, 24, 128)

    BB = 256
    while N % BB:
        BB //= 2
    grid = (N // BB,)

    full = lambda shape: pl.BlockSpec(shape, lambda i: tuple(0 for _ in shape))
    out = pl.pallas_call(
        _net_kernel,
        out_shape=jax.ShapeDtypeStruct((N, 10), jnp.float32),
        grid=grid,
        in_specs=[pl.BlockSpec((BB, 24, 128), lambda i: (i, 0, 0)),
                  full((3, 352, 256)), full((1, 128)),
                  full((896, 256)), full((1, 128)),
                  full((2, 256, 128)), full((128, 128)), full((1, 128)),
                  full((128, 128)), full((1, 128)),
                  full((128, 128)), full((1, 128))],
        out_specs=pl.BlockSpec((BB, 10), lambda i: (i, 0)),
        compiler_params=pltpu.CompilerParams(
            dimension_semantics=("parallel",)),
    )(x2, w1s, b1t, w2s, b2t, wf1a, wf1b, fc1_b, fc2_w, fc2_b,
      fc3_w, fc3_b)
    return out


# speedup vs baseline: 1.8732x; 1.0219x over previous
import functools
import math

import numpy as np
import jax
import jax.numpy as jnp
from jax.experimental import pallas as pl
from jax.experimental.pallas import tpu as pltpu


def _band(num_w: int, num_p: int, wp: int) -> np.ndarray:
    w = np.arange(num_w)[:, None, None]
    kj = np.arange(5)[None, :, None]
    p = np.arange(num_p)[None, None, :]
    return (w == 2 * p + wp + kj).astype(np.float32)


def _net_kernel(x_ref, w1_ref, b1_ref, w2_ref, b2_ref, wf1a_ref, wf1b_ref,
                bf1_ref, wf2_ref, bf2_ref, wf3_ref, bf3_ref, o_ref):
    def mm(a, w):
        return jax.lax.dot_general(a, w, (((1,), (0,)), ((), ())),
                                   preferred_element_type=jnp.float32)

    def conv2_row(oh2, rws):
        par, base = oh2 % 2, oh2 // 2
        start = 128 * (1 - par)
        acc = None
        for t in range(3):
            rhs = w2_ref[start + 256 * t: start + 256 * t + 256, :]
            m = mm(rws[base + t].astype(jnp.bfloat16), rhs)
            acc = m if acc is None else acc + m
        return acc

    def pool(ya, yb, bias):
        m = jnp.maximum(jnp.maximum(ya[:, :128], ya[:, 128:]),
                        jnp.maximum(yb[:, :128], yb[:, 128:]))
        return jnp.maximum(m + bias, 0.0)

    def conv1_row(oh):
        q, a = oh % 4, oh // 4
        acc = None
        for c in range(3):
            lhs = x_ref[:, c * 8 + a: c * 8 + a + 2, :]
            lhs = lhs.reshape(lhs.shape[0], 256).astype(jnp.bfloat16)
            t = mm(lhs, w1_ref[c, 96 - 32 * q: 352 - 32 * q, :])
            acc = t if acc is None else acc + t
        return acc

    b1 = b1_ref[...]
    p1 = [pool(conv1_row(2 * ph), conv1_row(2 * ph + 1), b1)
          for ph in range(14)]

    pairs = [jnp.concatenate([p1[2 * j], p1[2 * j + 1]], axis=1)
             for j in range(7)]

    b2 = b2_ref[...]
    p2 = [pool(conv2_row(2 * ph2, pairs),
               conv2_row(2 * ph2 + 1, pairs), b2)
          for ph2 in range(5)]

    q01 = jnp.concatenate([p2[0], p2[1]], axis=1)
    q23 = jnp.concatenate([p2[2], p2[3]], axis=1)
    h = mm(q01, wf1a_ref[0]) + mm(q23, wf1a_ref[1]) + mm(p2[4], wf1b_ref[...])
    h = jnp.maximum(h + bf1_ref[...], 0.0)
    h = jnp.maximum(mm(h, wf2_ref[...]) + bf2_ref[...], 0.0)
    o_ref[...] = (mm(h, wf3_ref[...]) + bf3_ref[...])[:, :10]


def _pack_weights(conv1_w, conv1_b, conv2_w, conv2_b, fc1_w, fc1_b):
    f32 = jnp.float32

    w1 = conv1_w.reshape(5, 5, 3, 128)[:, :, :, :8]
    ind1 = np.stack([_band(32, 14, 0), _band(32, 14, 1)])
    w1s = jnp.einsum('awkp,ikco->ciwapo', jnp.asarray(ind1), w1)
    w1s = w1s.reshape(3, 160, 2, 112)
    w1s = jnp.pad(w1s, ((0, 0), (96, 96), (0, 0), (0, 16)))
    w1s = w1s.reshape(3, 352, 256).astype(jnp.bfloat16)

    w2 = conv2_w.reshape(5, 5, 6, 128)[:, :, :, :16]
    w2p = jnp.pad(w2, ((0, 0), (0, 0), (0, 2), (0, 0)))
    ind2 = np.stack([_band(14, 5, 0), _band(14, 5, 1)])
    w2s = jnp.einsum('awkp,ikco->iwcapo', jnp.asarray(ind2), w2p)
    w2s = w2s.reshape(5, 14, 8, 2, 80)
    w2s = jnp.pad(w2s, ((0, 0), (0, 2), (0, 0), (0, 0), (0, 48)))
    w2s = w2s.reshape(5, 128, 256)
    w2s = jnp.pad(w2s, ((1, 1), (0, 0), (0, 0))).reshape(896, 256)
    w2s = w2s.astype(jnp.bfloat16)

    b1t = jnp.concatenate([jnp.tile(conv1_b[:, :8], (1, 14)),
                           jnp.zeros((1, 16), f32)], axis=1)
    b2t = jnp.concatenate([jnp.tile(conv2_b[:, :16], (1, 5)),
                           jnp.zeros((1, 48), f32)], axis=1)

    fr = fc1_w.reshape(5, 5, 128, 128)[:, :, :16, :]
    F = [jnp.pad(fr[p].reshape(80, 128), ((0, 48), (0, 0))) for p in range(5)]
    wf1a = jnp.stack([jnp.concatenate([F[0], F[1]], 0),
                      jnp.concatenate([F[2], F[3]], 0)])
    wf1b = F[4]
    return w1s, w2s, b1t, b2t, wf1a, wf1b


def kernel(conv1_w, conv1_b, conv2_w, conv2_b, fc1_w, fc1_b,
           fc2_w, fc2_b, fc3_w, fc3_b, x):
    N = x.shape[0]
    w1s, w2s, b1t, b2t, wf1a, wf1b = _pack_weights(
        conv1_w, conv1_b, conv2_w, conv2_b, fc1_w, fc1_b)

    x2 = x.reshape(N, 24, 128)

    BB = 256
    while N % BB:
        BB //= 2
    grid = (N // BB,)

    full = lambda shape: pl.BlockSpec(shape, lambda i: tuple(0 for _ in shape))
    out = pl.pallas_call(
        _net_kernel,
        out_shape=jax.ShapeDtypeStruct((N, 10), jnp.float32),
        grid=grid,
        in_specs=[pl.BlockSpec((BB, 24, 128), lambda i: (i, 0, 0)),
                  full((3, 352, 256)), full((1, 128)),
                  full((896, 256)), full((1, 128)),
                  full((2, 256, 128)), full((128, 128)), full((1, 128)),
                  full((128, 128)), full((1, 128)),
                  full((128, 128)), full((1, 128))],
        out_specs=pl.BlockSpec((BB, 10), lambda i: (i, 0)),
        compiler_params=pltpu.CompilerParams(
            dimension_semantics=("parallel",)),
    )(x2, w1s, b1t, w2s, b2t, wf1a, wf1b, fc1_b, fc2_w, fc2_b,
      fc3_w, fc3_b)
    return out
